# baseline (speedup 1.0000x reference)
import functools

import jax
import jax.numpy as jnp
import numpy as np
from jax import lax
from jax.experimental import pallas as pl
from jax.experimental.pallas import tpu as pltpu
from jax.experimental.pallas import tpu_sc as plsc

_L = 16
_MIN_I32 = np.int32(-2147483648)
_POS_MASK = np.int32(0x7FFFFFFF)


def _router_body(x_ref, w_ref, logit_ref):
    logit_ref[...] = lax.dot_general(
        x_ref[...], w_ref[...], (((1,), (0,)), ((), ())),
        preferred_element_type=jnp.float32)


def _router_logits(hidden2d, w_row, block_rows):
    n_rows, d = hidden2d.shape
    grid = (n_rows // block_rows,)
    return pl.pallas_call(
        _router_body,
        grid=grid,
        in_specs=[
            pl.BlockSpec((block_rows, d), lambda i: (i, 0)),
            pl.BlockSpec((d, 1), lambda i: (0, 0)),
        ],
        out_specs=pl.BlockSpec((block_rows, 1), lambda i: (i, 0)),
        out_shape=jax.ShapeDtypeStruct((n_rows, 1), jnp.float32),
        compiler_params=pltpu.CompilerParams(
            dimension_semantics=("arbitrary",),
        ),
    )(hidden2d, w_row)


def _make_route(batch, seq, cap):
    nv = seq // _L
    mesh = plsc.VectorSubcoreMesh(core_axis_name="c", subcore_axis_name="s")
    ncores = 2

    @functools.partial(
        pl.kernel,
        mesh=mesh,
        out_type=(
            jax.ShapeDtypeStruct((batch, cap), jnp.int32),
            jax.ShapeDtypeStruct((batch, seq - cap), jnp.int32),
        ),
        scratch_types=[
            pltpu.VMEM((seq,), jnp.float32),
            pltpu.VMEM((seq,), jnp.int32),
            pltpu.VMEM((256,), jnp.int32),
            pltpu.VMEM((cap,), jnp.int32),
            pltpu.VMEM((seq - cap,), jnp.int32),
        ],
        compiler_params=pltpu.CompilerParams(needs_layout_passes=False),
    )
    def route(logits_hbm, idx_hbm, cidx_hbm, lrow, kbuf, hist, idxbuf, cidxbuf):
        wid = lax.axis_index("s") * ncores + lax.axis_index("c")

        @pl.when(wid < batch)
        def _():
            b = wid
            pltpu.sync_copy(logits_hbm.at[b], lrow)
            iota = lax.iota(jnp.int32, _L)
            ones = jnp.ones((_L,), jnp.int32)

            def key_step(i, _):
                f = lrow[pl.ds(i * _L, _L)] + 0.0
                u = lax.bitcast_convert_type(f, jnp.int32)
                kbuf[pl.ds(i * _L, _L)] = jnp.where(u >= 0, u, u ^ _POS_MASK)
                return 0

            lax.fori_loop(0, nv, key_step, 0)

            pfx = jnp.int32(0)
            rem = jnp.int32(cap)
            for r in range(4):
                shift = 24 - 8 * r

                def zero_step(g, _):
                    hist[pl.ds(g * _L, _L)] = jnp.zeros((_L,), jnp.int32)
                    return 0

                lax.fori_loop(0, 256 // _L, zero_step, 0)

                def hist_step(i, _, pfx=pfx, shift=shift):
                    skey = kbuf[pl.ds(i * _L, _L)]
                    ukey = skey ^ _MIN_I32
                    t = lax.shift_right_logical(ukey, jnp.full_like(ukey, shift))
                    byte = t & 255
                    match = lax.shift_right_logical(t, jnp.full_like(t, 8)) == pfx
                    plsc.addupdate_scatter(hist, [byte], ones, mask=match)
                    return 0

                lax.fori_loop(0, nv, hist_step, 0)

                def scan_step(gi, carry, rem=rem):
                    c, vb, sg = carry
                    g = 256 // _L - 1 - gi
                    h = hist[pl.ds(g * _L, _L)]
                    ss_incl = lax.rev(plsc.cumsum(lax.rev(h, (0,))), (0,)) + c
                    ss_gt = ss_incl - h
                    cond = (ss_incl >= rem) & (ss_gt < rem)
                    vb = vb + jnp.sum(jnp.where(cond, iota + g * _L, 0))
                    sg = sg + jnp.sum(jnp.where(cond, ss_gt, 0))
                    return (c + jnp.sum(h), vb, sg)

                _, vbyte, ssgt = lax.fori_loop(
                    0, 256 // _L, scan_step,
                    (jnp.int32(0), jnp.int32(0), jnp.int32(0)))
                pfx = pfx * 256 + vbyte
                rem = rem - ssgt

            s_thr = pfx ^ _MIN_I32

            def sel_step(i, carry):
                ps, pu, ec = carry
                skey = kbuf[pl.ds(i * _L, _L)]
                m_gt = skey > s_thr
                m_eq = skey == s_thr
                ceq = plsc.cumsum(jnp.where(m_eq, 1, 0))
                msel = m_gt | (m_eq & ((ec + ceq) <= rem))
                idxv = iota + i * _L
                csel = plsc.cumsum(jnp.where(msel, 1, 0))
                plsc.store_scatter(idxbuf, [ps + csel - 1], idxv, mask=msel)
                mun = jnp.logical_not(msel)
                cun = plsc.cumsum(jnp.where(mun, 1, 0))
                plsc.store_scatter(cidxbuf, [pu + cun - 1], idxv, mask=mun)
                return (ps + jnp.sum(jnp.where(msel, 1, 0)),
                        pu + jnp.sum(jnp.where(mun, 1, 0)),
                        ec + jnp.sum(jnp.where(m_eq, 1, 0)))

            lax.fori_loop(0, nv, sel_step,
                          (jnp.int32(0), jnp.int32(0), jnp.int32(0)))
            pltpu.sync_copy(idxbuf, idx_hbm.at[b])
            pltpu.sync_copy(cidxbuf, cidx_hbm.at[b])

    return route


def _make_gather(batch, seq, d, cap, chunk=16):
    mesh = plsc.VectorSubcoreMesh(core_axis_name="c", subcore_axis_name="s")
    ncores = 2
    nw = 32
    rpt = batch * cap // nw
    tpb = nw // batch
    nch = rpt // chunk

    @functools.partial(
        pl.kernel,
        mesh=mesh,
        out_type=jax.ShapeDtypeStruct((batch * cap, d), jnp.float32),
        scratch_types=[
            pltpu.VMEM((nch, chunk), jnp.int32),
            pltpu.VMEM((chunk, d), jnp.float32),
            pltpu.VMEM((chunk, d), jnp.float32),
            pltpu.VMEM((chunk, d), jnp.float32),
            pltpu.SemaphoreType.DMA,
            pltpu.SemaphoreType.DMA,
            pltpu.SemaphoreType.DMA,
            pltpu.SemaphoreType.DMA,
            pltpu.SemaphoreType.DMA,
            pltpu.SemaphoreType.DMA,
        ],
        compiler_params=pltpu.CompilerParams(needs_layout_passes=False),
    )
    def gather(hidden_hbm, idx_hbm, sel_hbm, idx_v,
               buf0, buf1, buf2, g0, g1, g2, w0, w1, w2):
        wid = lax.axis_index("s") * ncores + lax.axis_index("c")
        b = wid // tpb
        base = (wid % tpb) * rpt
        for j in range(nch):
            pltpu.sync_copy(idx_hbm.at[b, pl.ds(base + j * chunk, chunk)],
                            idx_v.at[j])
        bufs = (buf0, buf1, buf2)
        gsems = (g0, g1, g2)
        wsems = (w0, w1, w2)

        def start_read(j):
            return pltpu.async_copy(
                hidden_hbm.at[b].at[idx_v.at[j]], bufs[j % 3], gsems[j % 3])

        def start_write(j):
            return pltpu.async_copy(
                bufs[j % 3],
                sel_hbm.at[pl.ds(wid * rpt + j * chunk, chunk)], wsems[j % 3])

        rd = {0: start_read(0)}
        if nch > 1:
            rd[1] = start_read(1)
        wr = {}
        for j in range(nch):
            rd[j].wait()
            wr[j] = start_write(j)
            if j + 2 < nch:
                if j - 1 >= 0:
                    wr[j - 1].wait()
                rd[j + 2] = start_read(j + 2)
        for j in range(max(0, nch - 3), nch):
            wr[j].wait()

    return gather


def _ffn_body(x_ref, w1_ref, w2_ref, out_ref):
    j = pl.program_id(1)

    @pl.when(j == 0)
    def _():
        out_ref[...] = jnp.zeros_like(out_ref)

    h = lax.dot_general(x_ref[...], w1_ref[...], (((1,), (0,)), ((), ())),
                        preferred_element_type=jnp.float32)
    hb = jax.nn.gelu(h).astype(w2_ref.dtype)
    out_ref[...] += lax.dot_general(hb, w2_ref[...], (((1,), (0,)), ((), ())),
                                    preferred_element_type=jnp.float32)


def _ffn(x, w1, w2, mb=1024, jb=512):
    m, d = x.shape
    dff = w1.shape[1]
    grid = (m // mb, dff // jb)
    return pl.pallas_call(
        _ffn_body,
        grid=grid,
        in_specs=[
            pl.BlockSpec((mb, d), lambda i, j: (i, 0)),
            pl.BlockSpec((d, jb), lambda i, j: (0, j)),
            pl.BlockSpec((jb, d), lambda i, j: (j, 0)),
        ],
        out_specs=pl.BlockSpec((mb, d), lambda i, j: (i, 0)),
        out_shape=jax.ShapeDtypeStruct((m, d), jnp.float32),
        compiler_params=pltpu.CompilerParams(
            dimension_semantics=("parallel", "arbitrary"),
        ),
    )(x, w1, w2)


def _make_scatter(batch, seq, d, cap, chunk=16):
    mesh = plsc.VectorSubcoreMesh(core_axis_name="c", subcore_axis_name="s")
    ncores = 2
    nw = 32
    rpt = batch * cap // nw
    rptu = batch * (seq - cap) // nw
    tpb = nw // batch
    nch = rpt // chunk
    nchu = rptu // chunk

    @functools.partial(
        pl.kernel,
        mesh=mesh,
        out_type=jax.ShapeDtypeStruct((batch, seq, d), jnp.float32),
        scratch_types=[
            pltpu.VMEM((nch, chunk), jnp.int32),
            pltpu.VMEM((nchu, chunk), jnp.int32),
            pltpu.VMEM((chunk, d), jnp.float32),
            pltpu.VMEM((chunk, d), jnp.float32),
            pltpu.VMEM((chunk, d), jnp.float32),
            pltpu.SemaphoreType.DMA,
            pltpu.SemaphoreType.DMA,
            pltpu.SemaphoreType.DMA,
            pltpu.SemaphoreType.DMA,
            pltpu.SemaphoreType.DMA,
            pltpu.SemaphoreType.DMA,
        ],
        compiler_params=pltpu.CompilerParams(needs_layout_passes=False),
    )
    def scatter(hidden_hbm, rows_hbm, idx_hbm, cidx_hbm, out_hbm,
                idx_v, cidx_v, buf0, buf1, buf2, g0, g1, g2, w0, w1, w2):
        wid = lax.axis_index("s") * ncores + lax.axis_index("c")
        b = wid // tpb
        base = (wid % tpb) * rpt
        baseu = (wid % tpb) * rptu
        for j in range(nch):
            pltpu.sync_copy(idx_hbm.at[b, pl.ds(base + j * chunk, chunk)],
                            idx_v.at[j])
        for j in range(nchu):
            pltpu.sync_copy(cidx_hbm.at[b, pl.ds(baseu + j * chunk, chunk)],
                            cidx_v.at[j])
        bufs = (buf0, buf1, buf2)
        gsems = (g0, g1, g2)
        wsems = (w0, w1, w2)

        def pipeline(n, start_read, start_write):
            rd = {0: start_read(0)}
            if n > 1:
                rd[1] = start_read(1)
            wr = {}
            for j in range(n):
                rd[j].wait()
                wr[j] = start_write(j)
                if j + 2 < n:
                    if j - 1 >= 0:
                        wr[j - 1].wait()
                    rd[j + 2] = start_read(j + 2)
            for j in range(max(0, n - 3), n):
                wr[j].wait()

        pipeline(
            nch,
            lambda j: pltpu.async_copy(
                rows_hbm.at[pl.ds(wid * rpt + j * chunk, chunk)],
                bufs[j % 3], gsems[j % 3]),
            lambda j: pltpu.async_copy(
                bufs[j % 3], out_hbm.at[b].at[idx_v.at[j]], wsems[j % 3]),
        )
        pipeline(
            nchu,
            lambda j: pltpu.async_copy(
                hidden_hbm.at[b].at[cidx_v.at[j]], bufs[j % 3], gsems[j % 3]),
            lambda j: pltpu.async_copy(
                bufs[j % 3], out_hbm.at[b].at[cidx_v.at[j]], wsems[j % 3]),
        )

    return scatter


def kernel(hidden_states, w_router, W1, W2):
    batch, seq, d = hidden_states.shape
    dff = W1.shape[1]
    cap = max(1, int(seq * 0.5))

    hidden2d = hidden_states.reshape(batch * seq, d)
    logits = _router_logits(hidden2d, w_router.reshape(d, 1),
                            block_rows=2048).reshape(batch, seq)

    idx, cidx = _make_route(batch, seq, cap)(logits)
    sel = _make_gather(batch, seq, d, cap)(hidden_states, idx)
    rows = _ffn(sel, W1, W2.astype(jnp.bfloat16))
    out = _make_scatter(batch, seq, d, cap)(hidden_states, rows, idx, cidx)
    return out


# speedup vs baseline: 1.0369x; 1.0369x over previous
import functools

import jax
import jax.numpy as jnp
import numpy as np
from jax import lax
from jax.experimental import pallas as pl
from jax.experimental.pallas import tpu as pltpu
from jax.experimental.pallas import tpu_sc as plsc

_L = 16
_MIN_I32 = np.int32(-2147483648)
_POS_MASK = np.int32(0x7FFFFFFF)


def _router_body(x_ref, w_ref, logit_ref):
    logit_ref[...] = lax.dot_general(
        x_ref[...], w_ref[...], (((1,), (0,)), ((), ())),
        preferred_element_type=jnp.float32)


def _router_logits(hidden2d, w_row, block_rows):
    n_rows, d = hidden2d.shape
    grid = (n_rows // block_rows,)
    return pl.pallas_call(
        _router_body,
        grid=grid,
        in_specs=[
            pl.BlockSpec((block_rows, d), lambda i: (i, 0)),
            pl.BlockSpec((d, 1), lambda i: (0, 0)),
        ],
        out_specs=pl.BlockSpec((block_rows, 1), lambda i: (i, 0)),
        out_shape=jax.ShapeDtypeStruct((n_rows, 1), jnp.float32),
        compiler_params=pltpu.CompilerParams(
            dimension_semantics=("arbitrary",),
        ),
    )(hidden2d, w_row)


def _make_route(batch, seq, cap):
    nv = seq // _L
    mesh = plsc.VectorSubcoreMesh(core_axis_name="c", subcore_axis_name="s")
    ncores = 2

    @functools.partial(
        pl.kernel,
        mesh=mesh,
        out_type=(
            jax.ShapeDtypeStruct((batch, cap), jnp.int32),
            jax.ShapeDtypeStruct((batch, seq - cap), jnp.int32),
        ),
        scratch_types=[
            pltpu.VMEM((seq,), jnp.float32),
            pltpu.VMEM((seq,), jnp.int32),
            pltpu.VMEM((256,), jnp.int32),
            pltpu.VMEM((cap,), jnp.int32),
            pltpu.VMEM((seq - cap,), jnp.int32),
        ],
        compiler_params=pltpu.CompilerParams(needs_layout_passes=False),
    )
    def route(logits_hbm, idx_hbm, cidx_hbm, lrow, kbuf, hist, idxbuf, cidxbuf):
        wid = lax.axis_index("s") * ncores + lax.axis_index("c")

        @pl.when(wid < batch)
        def _():
            b = wid
            pltpu.sync_copy(logits_hbm.at[b], lrow)
            iota = lax.iota(jnp.int32, _L)
            ones = jnp.ones((_L,), jnp.int32)

            def key_step(i, _):
                f = lrow[pl.ds(i * _L, _L)] + 0.0
                u = lax.bitcast_convert_type(f, jnp.int32)
                kbuf[pl.ds(i * _L, _L)] = jnp.where(u >= 0, u, u ^ _POS_MASK)
                return 0

            lax.fori_loop(0, nv, key_step, 0)

            pfx = jnp.int32(0)
            rem = jnp.int32(cap)
            for r in range(4):
                shift = 24 - 8 * r

                def zero_step(g, _):
                    hist[pl.ds(g * _L, _L)] = jnp.zeros((_L,), jnp.int32)
                    return 0

                lax.fori_loop(0, 256 // _L, zero_step, 0)

                def hist_step(i, _, pfx=pfx, shift=shift):
                    skey = kbuf[pl.ds(i * _L, _L)]
                    ukey = skey ^ _MIN_I32
                    t = lax.shift_right_logical(ukey, jnp.full_like(ukey, shift))
                    byte = t & 255
                    match = lax.shift_right_logical(t, jnp.full_like(t, 8)) == pfx
                    plsc.addupdate_scatter(hist, [byte], ones, mask=match)
                    return 0

                lax.fori_loop(0, nv, hist_step, 0)

                def scan_step(gi, carry, rem=rem):
                    c, vb, sg = carry
                    g = 256 // _L - 1 - gi
                    h = hist[pl.ds(g * _L, _L)]
                    ss_incl = lax.rev(plsc.cumsum(lax.rev(h, (0,))), (0,)) + c
                    ss_gt = ss_incl - h
                    cond = (ss_incl >= rem) & (ss_gt < rem)
                    vb = vb + jnp.sum(jnp.where(cond, iota + g * _L, 0))
                    sg = sg + jnp.sum(jnp.where(cond, ss_gt, 0))
                    return (c + jnp.sum(h), vb, sg)

                _, vbyte, ssgt = lax.fori_loop(
                    0, 256 // _L, scan_step,
                    (jnp.int32(0), jnp.int32(0), jnp.int32(0)))
                pfx = pfx * 256 + vbyte
                rem = rem - ssgt

            s_thr = pfx ^ _MIN_I32

            def sel_step(i, carry):
                ps, pu, ec = carry
                skey = kbuf[pl.ds(i * _L, _L)]
                m_gt = skey > s_thr
                m_eq = skey == s_thr
                ceq = plsc.cumsum(jnp.where(m_eq, 1, 0))
                msel = m_gt | (m_eq & ((ec + ceq) <= rem))
                idxv = iota + i * _L
                csel = plsc.cumsum(jnp.where(msel, 1, 0))
                plsc.store_scatter(idxbuf, [ps + csel - 1], idxv, mask=msel)
                mun = jnp.logical_not(msel)
                cun = plsc.cumsum(jnp.where(mun, 1, 0))
                plsc.store_scatter(cidxbuf, [pu + cun - 1], idxv, mask=mun)
                return (ps + jnp.sum(jnp.where(msel, 1, 0)),
                        pu + jnp.sum(jnp.where(mun, 1, 0)),
                        ec + jnp.sum(jnp.where(m_eq, 1, 0)))

            lax.fori_loop(0, nv, sel_step,
                          (jnp.int32(0), jnp.int32(0), jnp.int32(0)))
            pltpu.sync_copy(idxbuf, idx_hbm.at[b])
            pltpu.sync_copy(cidxbuf, cidx_hbm.at[b])

    return route


def _make_gather(batch, seq, d, cap, chunk=16):
    mesh = plsc.VectorSubcoreMesh(core_axis_name="c", subcore_axis_name="s")
    ncores = 2
    nw = 32
    rpt = batch * cap // nw
    tpb = nw // batch
    nch = rpt // chunk

    @functools.partial(
        pl.kernel,
        mesh=mesh,
        out_type=jax.ShapeDtypeStruct((batch * cap, d), jnp.float32),
        scratch_types=[
            pltpu.VMEM((nch, chunk), jnp.int32),
            pltpu.VMEM((chunk, d), jnp.float32),
            pltpu.VMEM((chunk, d), jnp.float32),
            pltpu.VMEM((chunk, d), jnp.float32),
            pltpu.SemaphoreType.DMA,
            pltpu.SemaphoreType.DMA,
            pltpu.SemaphoreType.DMA,
            pltpu.SemaphoreType.DMA,
            pltpu.SemaphoreType.DMA,
            pltpu.SemaphoreType.DMA,
        ],
        compiler_params=pltpu.CompilerParams(needs_layout_passes=False),
    )
    def gather(hidden_hbm, idx_hbm, sel_hbm, idx_v,
               buf0, buf1, buf2, g0, g1, g2, w0, w1, w2):
        wid = lax.axis_index("s") * ncores + lax.axis_index("c")
        b = wid // tpb
        base = (wid % tpb) * rpt
        for j in range(nch):
            pltpu.sync_copy(idx_hbm.at[b, pl.ds(base + j * chunk, chunk)],
                            idx_v.at[j])
        bufs = (buf0, buf1, buf2)
        gsems = (g0, g1, g2)
        wsems = (w0, w1, w2)

        def start_read(j):
            return pltpu.async_copy(
                hidden_hbm.at[b].at[idx_v.at[j]], bufs[j % 3], gsems[j % 3])

        def start_write(j):
            return pltpu.async_copy(
                bufs[j % 3],
                sel_hbm.at[pl.ds(wid * rpt + j * chunk, chunk)], wsems[j % 3])

        rd = {0: start_read(0)}
        if nch > 1:
            rd[1] = start_read(1)
        wr = {}
        for j in range(nch):
            rd[j].wait()
            wr[j] = start_write(j)
            if j + 2 < nch:
                if j - 1 >= 0:
                    wr[j - 1].wait()
                rd[j + 2] = start_read(j + 2)
        for j in range(max(0, nch - 3), nch):
            wr[j].wait()

    return gather


def _ffn_body(x_ref, w1_ref, w2_ref, out_ref):
    j = pl.program_id(1)

    @pl.when(j == 0)
    def _():
        out_ref[...] = jnp.zeros_like(out_ref)

    h = lax.dot_general(x_ref[...], w1_ref[...], (((1,), (0,)), ((), ())),
                        preferred_element_type=jnp.float32)
    hb = jax.nn.gelu(h).astype(w2_ref.dtype)
    out_ref[...] += lax.dot_general(hb, w2_ref[...], (((1,), (0,)), ((), ())),
                                    preferred_element_type=jnp.float32)


def _ffn(x, w1, w2, mb=1024, jb=512):
    m, d = x.shape
    dff = w1.shape[1]
    grid = (m // mb, dff // jb)
    return pl.pallas_call(
        _ffn_body,
        grid=grid,
        in_specs=[
            pl.BlockSpec((mb, d), lambda i, j: (i, 0)),
            pl.BlockSpec((d, jb), lambda i, j: (0, j)),
            pl.BlockSpec((jb, d), lambda i, j: (j, 0)),
        ],
        out_specs=pl.BlockSpec((mb, d), lambda i, j: (i, 0)),
        out_shape=jax.ShapeDtypeStruct((m, d), jnp.float32),
        compiler_params=pltpu.CompilerParams(
            dimension_semantics=("parallel", "arbitrary"),
        ),
    )(x, w1, w2)


def _make_scatter(batch, seq, d, cap, chunk=16):
    mesh = plsc.VectorSubcoreMesh(core_axis_name="c", subcore_axis_name="s")
    ncores = 2
    nw = 32
    rpt = batch * cap // nw
    rptu = batch * (seq - cap) // nw
    tpb = nw // batch
    nch = rpt // chunk
    nchu = rptu // chunk

    @functools.partial(
        pl.kernel,
        mesh=mesh,
        out_type=jax.ShapeDtypeStruct((batch, seq, d), jnp.float32),
        scratch_types=[
            pltpu.VMEM((nch, chunk), jnp.int32),
            pltpu.VMEM((nchu, chunk), jnp.int32),
            pltpu.VMEM((chunk, d), jnp.float32),
            pltpu.VMEM((chunk, d), jnp.float32),
            pltpu.VMEM((chunk, d), jnp.float32),
            pltpu.SemaphoreType.DMA,
            pltpu.SemaphoreType.DMA,
            pltpu.SemaphoreType.DMA,
            pltpu.SemaphoreType.DMA,
            pltpu.SemaphoreType.DMA,
            pltpu.SemaphoreType.DMA,
        ],
        compiler_params=pltpu.CompilerParams(needs_layout_passes=False),
    )
    def scatter(hidden_hbm, rows_hbm, idx_hbm, cidx_hbm, out_hbm,
                idx_v, cidx_v, buf0, buf1, buf2, g0, g1, g2, w0, w1, w2):
        wid = lax.axis_index("s") * ncores + lax.axis_index("c")
        b = wid // tpb
        base = (wid % tpb) * rpt
        baseu = (wid % tpb) * rptu
        for j in range(nch):
            pltpu.sync_copy(idx_hbm.at[b, pl.ds(base + j * chunk, chunk)],
                            idx_v.at[j])
        for j in range(nchu):
            pltpu.sync_copy(cidx_hbm.at[b, pl.ds(baseu + j * chunk, chunk)],
                            cidx_v.at[j])
        bufs = (buf0, buf1, buf2)
        gsems = (g0, g1, g2)
        wsems = (w0, w1, w2)

        def pipeline(n, start_read, start_write):
            rd = {0: start_read(0)}
            if n > 1:
                rd[1] = start_read(1)
            wr = {}
            for j in range(n):
                rd[j].wait()
                wr[j] = start_write(j)
                if j + 2 < n:
                    if j - 1 >= 0:
                        wr[j - 1].wait()
                    rd[j + 2] = start_read(j + 2)
            for j in range(max(0, n - 3), n):
                wr[j].wait()

        pipeline(
            nch,
            lambda j: pltpu.async_copy(
                rows_hbm.at[pl.ds(wid * rpt + j * chunk, chunk)],
                bufs[j % 3], gsems[j % 3]),
            lambda j: pltpu.async_copy(
                bufs[j % 3], out_hbm.at[b].at[idx_v.at[j]], wsems[j % 3]),
        )
        pipeline(
            nchu,
            lambda j: pltpu.async_copy(
                hidden_hbm.at[b].at[cidx_v.at[j]], bufs[j % 3], gsems[j % 3]),
            lambda j: pltpu.async_copy(
                bufs[j % 3], out_hbm.at[b].at[cidx_v.at[j]], wsems[j % 3]),
        )

    return scatter


def kernel(hidden_states, w_router, W1, W2):
    batch, seq, d = hidden_states.shape
    dff = W1.shape[1]
    cap = max(1, int(seq * 0.5))

    hidden2d = hidden_states.reshape(batch * seq, d)
    logits = _router_logits(hidden2d, w_router.reshape(d, 1),
                            block_rows=2048).reshape(batch, seq)

    idx, cidx = _make_route(batch, seq, cap)(logits)
    sel = _make_gather(batch, seq, d, cap)(hidden_states, idx)
    rows = _ffn(sel, W1, W2)
    out = _make_scatter(batch, seq, d, cap)(hidden_states, rows, idx, cidx)
    return out


# speedup vs baseline: 1.0382x; 1.0012x over previous
import functools

import jax
import jax.numpy as jnp
import numpy as np
from jax import lax
from jax.experimental import pallas as pl
from jax.experimental.pallas import tpu as pltpu
from jax.experimental.pallas import tpu_sc as plsc

_L = 16
_MIN_I32 = np.int32(-2147483648)
_POS_MASK = np.int32(0x7FFFFFFF)


def _router_body(x_ref, w_ref, logit_ref):
    logit_ref[...] = lax.dot_general(
        x_ref[...], w_ref[...], (((1,), (0,)), ((), ())),
        preferred_element_type=jnp.float32)


def _router_logits(hidden2d, w_row, block_rows):
    n_rows, d = hidden2d.shape
    grid = (n_rows // block_rows,)
    return pl.pallas_call(
        _router_body,
        grid=grid,
        in_specs=[
            pl.BlockSpec((block_rows, d), lambda i: (i, 0)),
            pl.BlockSpec((d, 1), lambda i: (0, 0)),
        ],
        out_specs=pl.BlockSpec((block_rows, 1), lambda i: (i, 0)),
        out_shape=jax.ShapeDtypeStruct((n_rows, 1), jnp.float32),
        compiler_params=pltpu.CompilerParams(
            dimension_semantics=("arbitrary",),
        ),
    )(hidden2d, w_row)


def _make_route(batch, seq, cap):
    nv = seq // _L
    mesh = plsc.VectorSubcoreMesh(core_axis_name="c", subcore_axis_name="s")
    ncores = 2

    @functools.partial(
        pl.kernel,
        mesh=mesh,
        out_type=(
            jax.ShapeDtypeStruct((batch, cap), jnp.int32),
            jax.ShapeDtypeStruct((batch, seq - cap), jnp.int32),
        ),
        scratch_types=[
            pltpu.VMEM((seq,), jnp.float32),
            pltpu.VMEM((seq,), jnp.int32),
            pltpu.VMEM((256,), jnp.int32),
            pltpu.VMEM((cap,), jnp.int32),
            pltpu.VMEM((seq - cap,), jnp.int32),
        ],
        compiler_params=pltpu.CompilerParams(needs_layout_passes=False),
    )
    def route(logits_hbm, idx_hbm, cidx_hbm, lrow, kbuf, hist, idxbuf, cidxbuf):
        wid = lax.axis_index("s") * ncores + lax.axis_index("c")

        @pl.when(wid < batch)
        def _():
            b = wid
            pltpu.sync_copy(logits_hbm.at[b], lrow)
            iota = lax.iota(jnp.int32, _L)
            ones = jnp.ones((_L,), jnp.int32)

            def key_step(i, _):
                f = lrow[pl.ds(i * _L, _L)] + 0.0
                u = lax.bitcast_convert_type(f, jnp.int32)
                kbuf[pl.ds(i * _L, _L)] = jnp.where(u >= 0, u, u ^ _POS_MASK)
                return 0

            lax.fori_loop(0, nv, key_step, 0)

            pfx = jnp.int32(0)
            rem = jnp.int32(cap)
            for r in range(4):
                shift = 24 - 8 * r

                def zero_step(g, _):
                    hist[pl.ds(g * _L, _L)] = jnp.zeros((_L,), jnp.int32)
                    return 0

                lax.fori_loop(0, 256 // _L, zero_step, 0)

                def hist_step(i, _, pfx=pfx, shift=shift):
                    skey = kbuf[pl.ds(i * _L, _L)]
                    ukey = skey ^ _MIN_I32
                    t = lax.shift_right_logical(ukey, jnp.full_like(ukey, shift))
                    byte = t & 255
                    match = lax.shift_right_logical(t, jnp.full_like(t, 8)) == pfx
                    plsc.addupdate_scatter(hist, [byte], ones, mask=match)
                    return 0

                lax.fori_loop(0, nv, hist_step, 0)

                def scan_step(gi, carry, rem=rem):
                    c, vb, sg = carry
                    g = 256 // _L - 1 - gi
                    h = hist[pl.ds(g * _L, _L)]
                    ss_incl = lax.rev(plsc.cumsum(lax.rev(h, (0,))), (0,)) + c
                    ss_gt = ss_incl - h
                    cond = (ss_incl >= rem) & (ss_gt < rem)
                    vb = vb + jnp.sum(jnp.where(cond, iota + g * _L, 0))
                    sg = sg + jnp.sum(jnp.where(cond, ss_gt, 0))
                    return (c + jnp.sum(h), vb, sg)

                _, vbyte, ssgt = lax.fori_loop(
                    0, 256 // _L, scan_step,
                    (jnp.int32(0), jnp.int32(0), jnp.int32(0)))
                pfx = pfx * 256 + vbyte
                rem = rem - ssgt

            s_thr = pfx ^ _MIN_I32

            def sel_step(i, carry):
                ps, pu, ec = carry
                skey = kbuf[pl.ds(i * _L, _L)]
                m_gt = skey > s_thr
                m_eq = skey == s_thr
                ceq = plsc.cumsum(jnp.where(m_eq, 1, 0))
                msel = m_gt | (m_eq & ((ec + ceq) <= rem))
                idxv = iota + i * _L
                csel = plsc.cumsum(jnp.where(msel, 1, 0))
                plsc.store_scatter(idxbuf, [ps + csel - 1], idxv, mask=msel)
                mun = jnp.logical_not(msel)
                cun = plsc.cumsum(jnp.where(mun, 1, 0))
                plsc.store_scatter(cidxbuf, [pu + cun - 1], idxv, mask=mun)
                return (ps + jnp.sum(jnp.where(msel, 1, 0)),
                        pu + jnp.sum(jnp.where(mun, 1, 0)),
                        ec + jnp.sum(jnp.where(m_eq, 1, 0)))

            lax.fori_loop(0, nv, sel_step,
                          (jnp.int32(0), jnp.int32(0), jnp.int32(0)))
            pltpu.sync_copy(idxbuf, idx_hbm.at[b])
            pltpu.sync_copy(cidxbuf, cidx_hbm.at[b])

    return route


def _make_gather(batch, seq, d, cap, chunk=16):
    mesh = plsc.VectorSubcoreMesh(core_axis_name="c", subcore_axis_name="s")
    ncores = 2
    nw = 32
    rpt = batch * cap // nw
    tpb = nw // batch
    nch = rpt // chunk

    @functools.partial(
        pl.kernel,
        mesh=mesh,
        out_type=jax.ShapeDtypeStruct((batch * cap, d), jnp.float32),
        scratch_types=[
            pltpu.VMEM((nch, chunk), jnp.int32),
            pltpu.VMEM((chunk, d), jnp.float32),
            pltpu.VMEM((chunk, d), jnp.float32),
            pltpu.VMEM((chunk, d), jnp.float32),
            pltpu.SemaphoreType.DMA,
            pltpu.SemaphoreType.DMA,
            pltpu.SemaphoreType.DMA,
            pltpu.SemaphoreType.DMA,
            pltpu.SemaphoreType.DMA,
            pltpu.SemaphoreType.DMA,
        ],
        compiler_params=pltpu.CompilerParams(needs_layout_passes=False),
    )
    def gather(hidden_hbm, idx_hbm, sel_hbm, idx_v,
               buf0, buf1, buf2, g0, g1, g2, w0, w1, w2):
        wid = lax.axis_index("s") * ncores + lax.axis_index("c")
        b = wid // tpb
        base = (wid % tpb) * rpt
        for j in range(nch):
            pltpu.sync_copy(idx_hbm.at[b, pl.ds(base + j * chunk, chunk)],
                            idx_v.at[j])
        bufs = (buf0, buf1, buf2)
        gsems = (g0, g1, g2)
        wsems = (w0, w1, w2)

        def start_read(j):
            return pltpu.async_copy(
                hidden_hbm.at[b].at[idx_v.at[j]], bufs[j % 3], gsems[j % 3])

        def start_write(j):
            return pltpu.async_copy(
                bufs[j % 3],
                sel_hbm.at[pl.ds(wid * rpt + j * chunk, chunk)], wsems[j % 3])

        rd = {0: start_read(0)}
        if nch > 1:
            rd[1] = start_read(1)
        wr = {}
        for j in range(nch):
            rd[j].wait()
            wr[j] = start_write(j)
            if j + 2 < nch:
                if j - 1 >= 0:
                    wr[j - 1].wait()
                rd[j + 2] = start_read(j + 2)
        for j in range(max(0, nch - 3), nch):
            wr[j].wait()

    return gather


def _ffn_body(x_ref, w1_ref, w2_ref, out_ref):
    j = pl.program_id(1)

    @pl.when(j == 0)
    def _():
        out_ref[...] = jnp.zeros_like(out_ref)

    h = lax.dot_general(x_ref[...], w1_ref[...], (((1,), (0,)), ((), ())),
                        preferred_element_type=jnp.float32)
    hb = jax.nn.gelu(h).astype(w2_ref.dtype)
    out_ref[...] += lax.dot_general(hb, w2_ref[...], (((1,), (0,)), ((), ())),
                                    preferred_element_type=jnp.float32)


def _ffn(x, w1, w2, mb=1024, jb=1024):
    m, d = x.shape
    dff = w1.shape[1]
    grid = (m // mb, dff // jb)
    return pl.pallas_call(
        _ffn_body,
        grid=grid,
        in_specs=[
            pl.BlockSpec((mb, d), lambda i, j: (i, 0)),
            pl.BlockSpec((d, jb), lambda i, j: (0, j)),
            pl.BlockSpec((jb, d), lambda i, j: (j, 0)),
        ],
        out_specs=pl.BlockSpec((mb, d), lambda i, j: (i, 0)),
        out_shape=jax.ShapeDtypeStruct((m, d), jnp.float32),
        compiler_params=pltpu.CompilerParams(
            dimension_semantics=("parallel", "arbitrary"),
            vmem_limit_bytes=110 * 1024 * 1024,
        ),
    )(x, w1, w2)


def _make_scatter(batch, seq, d, cap, chunk=16):
    mesh = plsc.VectorSubcoreMesh(core_axis_name="c", subcore_axis_name="s")
    ncores = 2
    nw = 32
    rpt = batch * cap // nw
    rptu = batch * (seq - cap) // nw
    tpb = nw // batch
    nch = rpt // chunk
    nchu = rptu // chunk

    @functools.partial(
        pl.kernel,
        mesh=mesh,
        out_type=jax.ShapeDtypeStruct((batch, seq, d), jnp.float32),
        scratch_types=[
            pltpu.VMEM((nch, chunk), jnp.int32),
            pltpu.VMEM((nchu, chunk), jnp.int32),
            pltpu.VMEM((chunk, d), jnp.float32),
            pltpu.VMEM((chunk, d), jnp.float32),
            pltpu.VMEM((chunk, d), jnp.float32),
            pltpu.SemaphoreType.DMA,
            pltpu.SemaphoreType.DMA,
            pltpu.SemaphoreType.DMA,
            pltpu.SemaphoreType.DMA,
            pltpu.SemaphoreType.DMA,
            pltpu.SemaphoreType.DMA,
        ],
        compiler_params=pltpu.CompilerParams(needs_layout_passes=False),
    )
    def scatter(hidden_hbm, rows_hbm, idx_hbm, cidx_hbm, out_hbm,
                idx_v, cidx_v, buf0, buf1, buf2, g0, g1, g2, w0, w1, w2):
        wid = lax.axis_index("s") * ncores + lax.axis_index("c")
        b = wid // tpb
        base = (wid % tpb) * rpt
        baseu = (wid % tpb) * rptu
        for j in range(nch):
            pltpu.sync_copy(idx_hbm.at[b, pl.ds(base + j * chunk, chunk)],
                            idx_v.at[j])
        for j in range(nchu):
            pltpu.sync_copy(cidx_hbm.at[b, pl.ds(baseu + j * chunk, chunk)],
                            cidx_v.at[j])
        bufs = (buf0, buf1, buf2)
        gsems = (g0, g1, g2)
        wsems = (w0, w1, w2)

        def pipeline(n, start_read, start_write):
            rd = {0: start_read(0)}
            if n > 1:
                rd[1] = start_read(1)
            wr = {}
            for j in range(n):
                rd[j].wait()
                wr[j] = start_write(j)
                if j + 2 < n:
                    if j - 1 >= 0:
                        wr[j - 1].wait()
                    rd[j + 2] = start_read(j + 2)
            for j in range(max(0, n - 3), n):
                wr[j].wait()

        pipeline(
            nch,
            lambda j: pltpu.async_copy(
                rows_hbm.at[pl.ds(wid * rpt + j * chunk, chunk)],
                bufs[j % 3], gsems[j % 3]),
            lambda j: pltpu.async_copy(
                bufs[j % 3], out_hbm.at[b].at[idx_v.at[j]], wsems[j % 3]),
        )
        pipeline(
            nchu,
            lambda j: pltpu.async_copy(
                hidden_hbm.at[b].at[cidx_v.at[j]], bufs[j % 3], gsems[j % 3]),
            lambda j: pltpu.async_copy(
                bufs[j % 3], out_hbm.at[b].at[cidx_v.at[j]], wsems[j % 3]),
        )

    return scatter


def kernel(hidden_states, w_router, W1, W2):
    batch, seq, d = hidden_states.shape
    dff = W1.shape[1]
    cap = max(1, int(seq * 0.5))

    hidden2d = hidden_states.reshape(batch * seq, d)
    logits = _router_logits(hidden2d, w_router.reshape(d, 1),
                            block_rows=2048).reshape(batch, seq)

    idx, cidx = _make_route(batch, seq, cap)(logits)
    sel = _make_gather(batch, seq, d, cap)(hidden_states, idx)
    rows = _ffn(sel, W1, W2.astype(jnp.bfloat16))
    out = _make_scatter(batch, seq, d, cap)(hidden_states, rows, idx, cidx)
    return out
